# Initial kernel scaffold; baseline (speedup 1.0000x reference)
import jax
import jax.numpy as jnp
from jax.experimental import pallas as pl


def kernel(inputs):
    raise NotImplementedError("write your pallas kernel here")


# speedup vs baseline: 8.4259x; 8.4259x over previous
import functools

import jax
import jax.numpy as jnp
from jax import lax
from jax.experimental import pallas as pl
from jax.experimental.pallas import tpu as pltpu
from jax.experimental.pallas import tpu_sc as plsc

MS = 512
NB = 64
N = MS * (MS + 1) // 2
NW = 32
BPW = NB // NW
RBLK = 64
NBLK = MS // RBLK
LANE = 16
NCH = MS // LANE


def _offset(r: int) -> int:
    return MS * r - r * (r - 1) // 2


_ALO = []
_SPAN = []
for _c in range(NBLK):
    _r0 = _c * RBLK
    _r1 = _r0 + RBLK - 1
    _lo = (_offset(_r0) - _r0) & ~7
    _lo = max(_lo, 0)
    _hi = _offset(_r1) + (MS - _r1)
    _span = -(-(_hi - _lo) // 16) * 16
    if _lo + _span > N:
        _lo = N - _span
    _ALO.append(_lo)
    _SPAN.append(_span)
_VIN = max(_SPAN)


def _body(x_hbm, out_hbm, vin, vout):
    cid = lax.axis_index("c")
    sid = lax.axis_index("s")
    wid = sid * 2 + cid

    lane = lax.iota(jnp.int32, LANE)

    for c in range(NBLK):
        r0 = c * RBLK
        alo = _ALO[c]
        span = _SPAN[c]

        def unit(lb, _, c=c, r0=r0, alo=alo, span=span):
            b = wid * BPW + lb
            pltpu.sync_copy(x_hbm.at[pl.ds(b * N + alo, span)],
                            vin.at[pl.ds(0, span)])

            def row(i, _, c=c, r0=r0, alo=alo):
                r = r0 + i
                base = (MS * r - ((r * (r - 1)) >> 1)) - r - alo
                zeros = jnp.zeros((LANE,), jnp.float32)
                for k in range(NCH):
                    if k < 4 * c:
                        vout[i, pl.ds(k * LANE, LANE)] = zeros
                    elif k < 4 * c + 4:
                        col = lane + (k * LANE)
                        keep = col >= r
                        g = plsc.load_gather(vin, [base + col], mask=keep)
                        vout[i, pl.ds(k * LANE, LANE)] = jnp.where(keep, g, 0.0)
                    else:
                        col = lane + (k * LANE)
                        g = plsc.load_gather(vin, [base + col])
                        vout[i, pl.ds(k * LANE, LANE)] = g
                return 0

            lax.fori_loop(0, RBLK, row, 0)
            pltpu.sync_copy(vout, out_hbm.at[b, pl.ds(r0, RBLK)])
            return 0

        lax.fori_loop(0, BPW, unit, 0)


def kernel(inputs):
    x_flat = inputs.reshape(NB * N)
    sc_kernel = pl.kernel(
        _body,
        out_type=jax.ShapeDtypeStruct((NB, MS, MS), jnp.float32),
        mesh=plsc.VectorSubcoreMesh(core_axis_name="c", subcore_axis_name="s"),
        scratch_types=[
            pltpu.VMEM((_VIN,), jnp.float32),
            pltpu.VMEM((RBLK, MS), jnp.float32),
        ],
        compiler_params=pltpu.CompilerParams(needs_layout_passes=False),
    )
    return sc_kernel(x_flat)


# speedup vs baseline: 11.1630x; 1.3248x over previous
import functools

import jax
import jax.numpy as jnp
from jax import lax
from jax.experimental import pallas as pl
from jax.experimental.pallas import tpu as pltpu
from jax.experimental.pallas import tpu_sc as plsc

MS = 512
NB = 64
N = MS * (MS + 1) // 2
NW = 32
BPW = NB // NW
RBLK = 64
NBLK = MS // RBLK
LANE = 16
NCH = MS // LANE


def _offset(r: int) -> int:
    return MS * r - r * (r - 1) // 2


_ALO = []
_SPAN = []
for _c in range(NBLK):
    _r0 = _c * RBLK
    _r1 = _r0 + RBLK - 1
    _lo = (_offset(_r0) - _r0) & ~7
    _lo = max(_lo, 0)
    _hi = _offset(_r1) + (MS - _r1)
    _span = -(-(_hi - _lo) // 16) * 16
    if _lo + _span > N:
        _lo = N - _span
    _ALO.append(_lo)
    _SPAN.append(_span)
_VIN = max(_SPAN)


def _body(x_hbm, out_hbm, vin0, vin1, vout0, vout1,
          in_sem0, in_sem1, out_sem0, out_sem1):
    cid = lax.axis_index("c")
    sid = lax.axis_index("s")
    wid = sid * 2 + cid

    lane = lax.iota(jnp.int32, LANE)
    vins = [vin0, vin1]
    vouts = [vout0, vout1]
    in_sems = [in_sem0, in_sem1]
    out_sems = [out_sem0, out_sem1]

    units = [(c, lb) for c in range(NBLK) for lb in range(BPW)]

    def start_in(u):
        c, lb = units[u]
        p = lb
        b = wid * BPW + lb
        alo, span = _ALO[c], _SPAN[c]
        return pltpu.async_copy(
            x_hbm.at[pl.ds(b * N + alo, span)],
            vins[p].at[pl.ds(0, span)], in_sems[p])

    in_dma = [None] * len(units)
    out_dma = [None] * len(units)
    in_dma[0] = start_in(0)
    if len(units) > 1:
        in_dma[1] = start_in(1)

    for u, (c, lb) in enumerate(units):
        p = lb
        r0 = c * RBLK
        alo = _ALO[c]
        vin = vins[p]
        vout = vouts[p]

        in_dma[u].wait()
        if u >= 2:
            out_dma[u - 2].wait()
        if u >= 1 and u + 1 < len(units):
            in_dma[u + 1] = start_in(u + 1)

        def row(i, _, c=c, r0=r0, alo=alo, vin=vin, vout=vout):
            r = r0 + i
            base = (MS * r - ((r * (r - 1)) >> 1)) - r - alo
            zeros = jnp.zeros((LANE,), jnp.float32)
            for k in range(max(0, 4 * c - 4), NCH):
                if k < 4 * c:
                    vout[i, pl.ds(k * LANE, LANE)] = zeros
                elif k < 4 * c + 4:
                    col = lane + (k * LANE)
                    keep = col >= r
                    g = plsc.load_gather(vin, [base + col], mask=keep)
                    vout[i, pl.ds(k * LANE, LANE)] = jnp.where(keep, g, 0.0)
                else:
                    col = lane + (k * LANE)
                    g = plsc.load_gather(vin, [base + col])
                    vout[i, pl.ds(k * LANE, LANE)] = g
            return 0

        lax.fori_loop(0, RBLK, row, 0)
        b = wid * BPW + lb
        out_dma[u] = pltpu.async_copy(
            vout, out_hbm.at[b, pl.ds(r0, RBLK)], out_sems[p])

    out_dma[-2].wait()
    out_dma[-1].wait()


def kernel(inputs):
    x_flat = inputs.reshape(NB * N)
    sc_kernel = pl.kernel(
        _body,
        out_type=jax.ShapeDtypeStruct((NB, MS, MS), jnp.float32),
        mesh=plsc.VectorSubcoreMesh(core_axis_name="c", subcore_axis_name="s"),
        scratch_types=[
            pltpu.VMEM((_VIN,), jnp.float32),
            pltpu.VMEM((_VIN,), jnp.float32),
            pltpu.VMEM((RBLK, MS), jnp.float32),
            pltpu.VMEM((RBLK, MS), jnp.float32),
            pltpu.SemaphoreType.DMA,
            pltpu.SemaphoreType.DMA,
            pltpu.SemaphoreType.DMA,
            pltpu.SemaphoreType.DMA,
        ],
        compiler_params=pltpu.CompilerParams(needs_layout_passes=False),
    )
    return sc_kernel(x_flat)


# speedup vs baseline: 16.0423x; 1.4371x over previous
import functools

import jax
import jax.numpy as jnp
from jax import lax
from jax.experimental import pallas as pl
from jax.experimental.pallas import tpu as pltpu
from jax.experimental.pallas import tpu_sc as plsc

MS = 512
NB = 64
N = MS * (MS + 1) // 2
NW = 32
BPW = NB // NW
RBLK = 64
NBLK = MS // RBLK
LANE = 16
NCH = MS // LANE


def _offset(r: int) -> int:
    return MS * r - r * (r - 1) // 2


_ALO = []
_SPAN = []
for _c in range(NBLK):
    _r0 = _c * RBLK
    _r1 = _r0 + RBLK - 1
    _lo = (_offset(_r0) - _r0) & ~7
    _lo = max(_lo, 0)
    _hi = _offset(_r1) + (MS - _r1)
    _span = -(-(_hi - _lo) // 16) * 16
    if _lo + _span > N:
        _lo = N - _span
    _ALO.append(_lo)
    _SPAN.append(_span)
_VIN = max(_SPAN)


def _body(x_hbm, out_hbm, vin0, vin1, vout0, vout1,
          in_sem0, in_sem1, out_sem0, out_sem1):
    cid = lax.axis_index("c")
    sid = lax.axis_index("s")
    wid = sid * 2 + cid

    lane = lax.iota(jnp.int32, LANE)
    vins = [vin0, vin1]
    vouts = [vout0, vout1]
    in_sems = [in_sem0, in_sem1]
    out_sems = [out_sem0, out_sem1]

    units = [(c, lb) for c in range(NBLK) for lb in range(BPW)]

    def start_in(u):
        c, lb = units[u]
        p = lb
        b = wid * BPW + lb
        alo, span = _ALO[c], _SPAN[c]
        return pltpu.async_copy(
            x_hbm.at[pl.ds(b * N + alo, span)],
            vins[p].at[pl.ds(0, span)], in_sems[p])

    in_dma = [None] * len(units)
    out_dma = [None] * len(units)
    in_dma[0] = start_in(0)
    if len(units) > 1:
        in_dma[1] = start_in(1)

    for u, (c, lb) in enumerate(units):
        p = lb
        r0 = c * RBLK
        alo = _ALO[c]
        vin = vins[p]
        vout = vouts[p]

        in_dma[u].wait()
        if u >= 2:
            out_dma[u - 2].wait()
        if u >= 1 and u + 1 < len(units):
            in_dma[u + 1] = start_in(u + 1)

        @plsc.parallel_loop(0, RBLK, 1, unroll=2)
        def row(i, c=c, r0=r0, alo=alo, vin=vin, vout=vout):
            r = r0 + i
            base = (MS * r - ((r * (r - 1)) >> 1)) - r - alo
            zeros = jnp.zeros((LANE,), jnp.float32)
            for k in range(max(0, 4 * c - 4), NCH):
                if k < 4 * c:
                    vout[i, pl.ds(k * LANE, LANE)] = zeros
                elif k < 4 * c + 4:
                    col = lane + (k * LANE)
                    keep = col >= r
                    g = plsc.load_gather(vin, [base + col], mask=keep)
                    vout[i, pl.ds(k * LANE, LANE)] = jnp.where(keep, g, 0.0)
                else:
                    col = lane + (k * LANE)
                    g = plsc.load_gather(vin, [base + col])
                    vout[i, pl.ds(k * LANE, LANE)] = g
        b = wid * BPW + lb
        out_dma[u] = pltpu.async_copy(
            vout, out_hbm.at[b, pl.ds(r0, RBLK)], out_sems[p])

    out_dma[-2].wait()
    out_dma[-1].wait()


def kernel(inputs):
    x_flat = inputs.reshape(NB * N)
    sc_kernel = pl.kernel(
        _body,
        out_type=jax.ShapeDtypeStruct((NB, MS, MS), jnp.float32),
        mesh=plsc.VectorSubcoreMesh(core_axis_name="c", subcore_axis_name="s"),
        scratch_types=[
            pltpu.VMEM((_VIN,), jnp.float32),
            pltpu.VMEM((_VIN,), jnp.float32),
            pltpu.VMEM((RBLK, MS), jnp.float32),
            pltpu.VMEM((RBLK, MS), jnp.float32),
            pltpu.SemaphoreType.DMA,
            pltpu.SemaphoreType.DMA,
            pltpu.SemaphoreType.DMA,
            pltpu.SemaphoreType.DMA,
        ],
        compiler_params=pltpu.CompilerParams(needs_layout_passes=False),
    )
    return sc_kernel(x_flat)
